# baseline (speedup 1.0000x reference)
import functools

import jax
import jax.numpy as jnp
from jax import lax
from jax.experimental import pallas as pl
from jax.experimental.pallas import tpu as pltpu
from jax.experimental.pallas import tpu_sc as plsc

N = 8388608
NSEG = 64
L = 16
NC, NS = 2, 16
NW = NC * NS
PER_W = N // NW
CHUNK = 4096
N_CHUNKS = PER_W // CHUNK
NBUF = 8
ROUNDS = N_CHUNKS // NBUF

_LN2 = 0.6931471805599453


def _ln1p_small(u):
    t = u / (u + 2.0)
    s = t * t
    return 2.0 * t * (1.0 + s * (1.0 / 3.0 + s * (0.2 + s * (1.0 / 7.0 + s * (1.0 / 9.0)))))


def _ln_pos(y):
    bits = plsc.bitcast(y, jnp.int32)
    e = ((bits >> 23) - 127).astype(jnp.float32)
    m = plsc.bitcast((bits & 0x007FFFFF) | 0x3F800000, jnp.float32)
    return e * _LN2 + _ln1p_small(m - 1.0)


_mesh = plsc.VectorSubcoreMesh(core_axis_name="c", subcore_axis_name="s")


@functools.partial(
    pl.kernel,
    out_type=jax.ShapeDtypeStruct((N,), jnp.float32),
    mesh=_mesh,
    compiler_params=pltpu.CompilerParams(needs_layout_passes=False),
    scratch_types=(
        [pltpu.VMEM((NSEG,), jnp.float32)] * 2
        + [pltpu.VMEM((CHUNK,), jnp.float32)] * (2 * NBUF)
        + [pltpu.SemaphoreType.DMA] * (2 * NBUF)
    ),
)
def _sc_flow(x_hbm, raw_hbm, out_hbm, raw_v, tab_v, *rest):
    xbuf = rest[0:NBUF]
    obuf = rest[NBUF:2 * NBUF]
    sem_in = rest[2 * NBUF:3 * NBUF]
    sem_out = rest[3 * NBUF:4 * NBUF]
    wid = lax.axis_index("s") * NC + lax.axis_index("c")
    base = wid * PER_W

    copies_in = [None] * NBUF
    copies_out = [None] * NBUF
    for c in range(NBUF):
        off = base + c * CHUNK
        copies_in[c] = pltpu.async_copy(
            x_hbm.at[pl.ds(off, CHUNK)], xbuf[c], sem_in[c])

    pltpu.sync_copy(raw_hbm, raw_v)
    hs = []
    total = jnp.float32(0.0)
    for g in range(NSEG // L):
        r = raw_v[pl.ds(g * L, L)]
        u = jnp.exp(-jnp.abs(r))
        sp = jnp.maximum(r, 0.0) + _ln1p_small(u)
        h = sp + 1e-6
        hs.append(h)
        total = total + jnp.sum(h)
    scale = float(NSEG) / lax.broadcast(total, (L,))
    for g in range(NSEG // L):
        y = hs[g] * scale + 1e-10
        tab_v[pl.ds(g * L, L)] = _ln_pos(y)

    def round_body(r, carry):
        for j in range(NBUF):
            c = r * NBUF + j
            off = base + c * CHUNK
            xb = xbuf[j]
            ob = obuf[j]
            pltpu.make_async_copy(
                x_hbm.at[pl.ds(off, CHUNK)], xb, sem_in[j]).wait()

            @pl.when(r > 0)
            def _(ob=ob, off=off, j=j):
                pltpu.make_async_copy(
                    ob, out_hbm.at[pl.ds(off, CHUNK)], sem_out[j]).wait()

            @plsc.parallel_loop(0, CHUNK // L, 1, unroll=16)
            def _(i, xb=xb, ob=ob):
                xv = xb[pl.ds(i * L, L)]
                idx = (xv * float(NSEG)).astype(jnp.int32)
                idx = jnp.minimum(jnp.maximum(idx, 0), NSEG - 1)
                ob[pl.ds(i * L, L)] = plsc.load_gather(tab_v, [idx])

            pltpu.async_copy(ob, out_hbm.at[pl.ds(off, CHUNK)], sem_out[j])

            @pl.when(c + NBUF < N_CHUNKS)
            def _(xb=xb, off=off, j=j):
                pltpu.async_copy(
                    x_hbm.at[pl.ds(off + NBUF * CHUNK, CHUNK)], xb, sem_in[j])

        return carry

    lax.fori_loop(0, ROUNDS, round_body, 0)

    for b in range(NBUF):
        off = base + (N_CHUNKS - NBUF + b) * CHUNK
        pltpu.make_async_copy(
            obuf[b], out_hbm.at[pl.ds(off, CHUNK)], sem_out[b]).wait()


def kernel(x, raw):
    return _sc_flow(x, raw)


# speedup vs baseline: 1.0169x; 1.0169x over previous
import functools

import jax
import jax.numpy as jnp
from jax import lax
from jax.experimental import pallas as pl
from jax.experimental.pallas import tpu as pltpu
from jax.experimental.pallas import tpu_sc as plsc

N = 8388608
NSEG = 64
L = 16
NC, NS = 2, 16
NW = NC * NS
PER_W = N // NW
CHUNK = 16384
N_CHUNKS = PER_W // CHUNK
NBUF = 4
ROUNDS = N_CHUNKS // NBUF
PREF = 2

_LN2 = 0.6931471805599453


def _ln1p_small(u):
    t = u / (u + 2.0)
    s = t * t
    return 2.0 * t * (1.0 + s * (1.0 / 3.0 + s * (0.2 + s * (1.0 / 7.0 + s * (1.0 / 9.0)))))


def _ln_pos(y):
    bits = plsc.bitcast(y, jnp.int32)
    e = ((bits >> 23) - 127).astype(jnp.float32)
    m = plsc.bitcast((bits & 0x007FFFFF) | 0x3F800000, jnp.float32)
    return e * _LN2 + _ln1p_small(m - 1.0)


_mesh = plsc.VectorSubcoreMesh(core_axis_name="c", subcore_axis_name="s")


@functools.partial(
    pl.kernel,
    out_type=jax.ShapeDtypeStruct((N,), jnp.float32),
    mesh=_mesh,
    compiler_params=pltpu.CompilerParams(needs_layout_passes=False),
    scratch_types=(
        [pltpu.VMEM((NSEG,), jnp.float32)] * 2
        + [pltpu.VMEM((CHUNK,), jnp.float32)] * NBUF
        + [pltpu.SemaphoreType.DMA] * (2 * NBUF)
    ),
)
def _sc_flow(x_hbm, raw_hbm, out_hbm, raw_v, tab_v, *rest):
    bufs = rest[0:NBUF]
    sem_in = rest[NBUF:2 * NBUF]
    sem_out = rest[2 * NBUF:3 * NBUF]
    wid = lax.axis_index("s") * NC + lax.axis_index("c")
    base = wid * PER_W

    for c in range(PREF):
        off = base + c * CHUNK
        pltpu.async_copy(x_hbm.at[pl.ds(off, CHUNK)], bufs[c], sem_in[c])

    pltpu.sync_copy(raw_hbm, raw_v)
    hs = []
    total = jnp.float32(0.0)
    for g in range(NSEG // L):
        r = raw_v[pl.ds(g * L, L)]
        u = jnp.exp(-jnp.abs(r))
        sp = jnp.maximum(r, 0.0) + _ln1p_small(u)
        h = sp + 1e-6
        hs.append(h)
        total = total + jnp.sum(h)
    scale = float(NSEG) / lax.broadcast(total, (L,))
    for g in range(NSEG // L):
        y = hs[g] * scale + 1e-10
        tab_v[pl.ds(g * L, L)] = _ln_pos(y)

    def round_body(r, carry):
        for j in range(NBUF):
            c = r * NBUF + j
            off = base + c * CHUNK
            bb = bufs[j]
            pltpu.make_async_copy(
                x_hbm.at[pl.ds(off, CHUNK)], bb, sem_in[j]).wait()

            @plsc.parallel_loop(0, CHUNK // L, 1, unroll=16)
            def _(i, bb=bb):
                xv = bb[pl.ds(i * L, L)]
                idx = (xv * float(NSEG)).astype(jnp.int32)
                idx = jnp.minimum(jnp.maximum(idx, 0), NSEG - 1)
                bb[pl.ds(i * L, L)] = plsc.load_gather(tab_v, [idx])

            pltpu.async_copy(bb, out_hbm.at[pl.ds(off, CHUNK)], sem_out[j])

            j2 = (j + PREF) % NBUF
            b2 = bufs[j2]

            @pl.when(c + PREF < N_CHUNKS)
            def _(b2=b2, off=off, j2=j2, c=c):
                @pl.when(c + PREF - NBUF >= 0)
                def _():
                    pltpu.make_async_copy(
                        b2, out_hbm.at[pl.ds(off, CHUNK)], sem_out[j2]).wait()
                pltpu.async_copy(
                    x_hbm.at[pl.ds(off + PREF * CHUNK, CHUNK)], b2, sem_in[j2])

        return carry

    lax.fori_loop(0, ROUNDS, round_body, 0)

    for c in range(N_CHUNKS - NBUF, N_CHUNKS):
        off = base + c * CHUNK
        pltpu.make_async_copy(
            bufs[c % NBUF], out_hbm.at[pl.ds(off, CHUNK)], sem_out[c % NBUF]).wait()


def kernel(x, raw):
    return _sc_flow(x, raw)


# speedup vs baseline: 1.0638x; 1.0461x over previous
import functools

import jax
import jax.numpy as jnp
from jax import lax
from jax.experimental import pallas as pl
from jax.experimental.pallas import tpu as pltpu
from jax.experimental.pallas import tpu_sc as plsc

N = 8388608
NSEG = 64
L = 16
NC, NS = 2, 16
NW = NC * NS
PER_W = N // NW
CHUNK = 8192
N_CHUNKS = PER_W // CHUNK
NBUF = 4
ROUNDS = N_CHUNKS // NBUF

_LN2 = 0.6931471805599453


def _ln1p_small(u):
    t = u / (u + 2.0)
    s = t * t
    return 2.0 * t * (1.0 + s * (1.0 / 3.0 + s * (0.2 + s * (1.0 / 7.0 + s * (1.0 / 9.0)))))


def _ln_pos(y):
    bits = plsc.bitcast(y, jnp.int32)
    e = ((bits >> 23) - 127).astype(jnp.float32)
    m = plsc.bitcast((bits & 0x007FFFFF) | 0x3F800000, jnp.float32)
    return e * _LN2 + _ln1p_small(m - 1.0)


_mesh = plsc.VectorSubcoreMesh(core_axis_name="c", subcore_axis_name="s")


@functools.partial(
    pl.kernel,
    out_type=jax.ShapeDtypeStruct((N,), jnp.float32),
    mesh=_mesh,
    compiler_params=pltpu.CompilerParams(needs_layout_passes=False),
    scratch_types=(
        [pltpu.VMEM((NSEG,), jnp.float32)] * 2
        + [pltpu.VMEM((CHUNK,), jnp.float32)] * (2 * NBUF)
        + [pltpu.SemaphoreType.DMA] * (2 * NBUF)
    ),
)
def _sc_flow(x_hbm, raw_hbm, out_hbm, raw_v, tab_v, *rest):
    xbuf = rest[0:NBUF]
    obuf = rest[NBUF:2 * NBUF]
    sem_in = rest[2 * NBUF:3 * NBUF]
    sem_out = rest[3 * NBUF:4 * NBUF]
    wid = lax.axis_index("s") * NC + lax.axis_index("c")
    base = wid * PER_W

    copies_in = [None] * NBUF
    copies_out = [None] * NBUF
    for c in range(NBUF):
        off = base + c * CHUNK
        copies_in[c] = pltpu.async_copy(
            x_hbm.at[pl.ds(off, CHUNK)], xbuf[c], sem_in[c])

    pltpu.sync_copy(raw_hbm, raw_v)
    hs = []
    total = jnp.float32(0.0)
    for g in range(NSEG // L):
        r = raw_v[pl.ds(g * L, L)]
        u = jnp.exp(-jnp.abs(r))
        sp = jnp.maximum(r, 0.0) + _ln1p_small(u)
        h = sp + 1e-6
        hs.append(h)
        total = total + jnp.sum(h)
    scale = float(NSEG) / lax.broadcast(total, (L,))
    for g in range(NSEG // L):
        y = hs[g] * scale + 1e-10
        tab_v[pl.ds(g * L, L)] = _ln_pos(y)

    def round_body(r, carry):
        for j in range(NBUF):
            c = r * NBUF + j
            off = base + c * CHUNK
            xb = xbuf[j]
            ob = obuf[j]
            pltpu.make_async_copy(
                x_hbm.at[pl.ds(off, CHUNK)], xb, sem_in[j]).wait()

            @pl.when(r > 0)
            def _(ob=ob, off=off, j=j):
                pltpu.make_async_copy(
                    ob, out_hbm.at[pl.ds(off, CHUNK)], sem_out[j]).wait()

            @plsc.parallel_loop(0, CHUNK // L, 1, unroll=16)
            def _(i, xb=xb, ob=ob):
                xv = xb[pl.ds(i * L, L)]
                idx = (xv * float(NSEG)).astype(jnp.int32)
                idx = jnp.minimum(jnp.maximum(idx, 0), NSEG - 1)
                ob[pl.ds(i * L, L)] = plsc.load_gather(tab_v, [idx])

            pltpu.async_copy(ob, out_hbm.at[pl.ds(off, CHUNK)], sem_out[j])

            @pl.when(c + NBUF < N_CHUNKS)
            def _(xb=xb, off=off, j=j):
                pltpu.async_copy(
                    x_hbm.at[pl.ds(off + NBUF * CHUNK, CHUNK)], xb, sem_in[j])

        return carry

    lax.fori_loop(0, ROUNDS, round_body, 0)

    for b in range(NBUF):
        off = base + (N_CHUNKS - NBUF + b) * CHUNK
        pltpu.make_async_copy(
            obuf[b], out_hbm.at[pl.ds(off, CHUNK)], sem_out[b]).wait()


def kernel(x, raw):
    return _sc_flow(x, raw)
